# baseline (speedup 1.0000x reference)
import functools

import jax
import jax.numpy as jnp
from jax import lax
from jax.experimental import pallas as pl
from jax.experimental.pallas import tpu as pltpu
from jax.experimental.pallas import tpu_sc as plsc

NUM_EMBEDDINGS = 1000000
EMBEDDING_DIM = 64

NBATCH = 4096
NSEQ = 50
NUM_CORES = 2
NUM_SUBCORES = 16
NW = NUM_CORES * NUM_SUBCORES

SLAB_V = 384
NSLAB = NUM_EMBEDDINGS // SLAB_V
TAIL_V = NUM_EMBEDDINGS - NSLAB * SLAB_V
RUNROLL = 2


def _shuffle(src, out_v, ok, nrows):
    iota = lax.iota(jnp.int32, 16)
    perms = [(iota + d) & 15 for d in range(16)]
    out2d = out_v.at[ok]

    @plsc.parallel_loop(0, (nrows // 16) * 8, 1, unroll=RUNROLL)
    def _(blk):
        rg = blk >> 3
        fg = (blk >> 1) & 3
        p = blk & 1
        r0 = rg * 16
        f0 = fg * 16
        rowr = iota + f0
        colw = iota + (p * 64 + f0)
        base = 2 * r0 + p
        for d in range(16):
            colr = 2 * perms[d] + base
            x = plsc.load_gather(src, [rowr, colr])
            plsc.store_scatter(out2d, [perms[d] + r0, colw], x)


RING = 2


def _transpose_body(wt_hbm, tail_hbm, wp_hbm, in_v, out_v, *sems):
    isems = sems[:RING]
    osems = sems[RING:]
    wid = lax.axis_index("s") * NUM_CORES + lax.axis_index("c")

    def in_start(j, k):
        pltpu.async_copy(
            wt_hbm.at[:, pl.ds(j * SLAB_V, SLAB_V)], in_v.at[k], isems[k]
        )

    def in_wait(k):
        pltpu.make_async_copy(
            wt_hbm.at[:, pl.ds(0, SLAB_V)], in_v.at[k], isems[k]
        ).wait()

    def out_start(j, k):
        pltpu.async_copy(
            out_v.at[k], wp_hbm.at[pl.ds(j * (SLAB_V // 2), SLAB_V // 2)], osems[k]
        )

    def out_wait(k):
        pltpu.make_async_copy(
            out_v.at[k], wp_hbm.at[pl.ds(0, SLAB_V // 2)], osems[k]
        ).wait()

    def do_slab(j, k, first, last):
        in_wait(k)

        @pl.when(jnp.logical_not(first))
        def _():
            out_wait(k)

        _shuffle(in_v.at[k], out_v, k, SLAB_V // 2)

        @pl.when(jnp.logical_not(last))
        def _():
            in_start(j + RING * NW, k)

        out_start(j, k)

    nmine = (NSLAB - 1 - wid) // NW + 1

    for k in range(RING):
        in_start(wid + k * NW, k)

    def group(i, carry):
        for k in range(RING):
            idx = RING * i + k

            @pl.when(idx < nmine)
            def _():
                do_slab(wid + idx * NW, k, idx < RING, idx + RING >= nmine)

        return carry

    lax.fori_loop(0, (nmine + RING - 1) // RING, group, 0)
    for k in range(RING):
        out_wait(k)

    @pl.when(wid == 0)
    def _():
        pltpu.sync_copy(tail_hbm, in_v.at[0, :, pl.ds(0, 128)])
        _shuffle(in_v.at[0], out_v, 0, TAIL_V // 2)
        pltpu.sync_copy(
            out_v.at[0, pl.ds(0, TAIL_V // 2)],
            wp_hbm.at[pl.ds(NSLAB * SLAB_V // 2, TAIL_V // 2)],
        )


COLS_PER_W = NBATCH // NW
NBUF = 5
NOUT = 5
INNER = 5
NGROUPS = NSEQ // INNER


def _gather_body(idx_hbm, table_hbm, out_hbm, idx_v, row_v, out_v, *sems):
    gsems = sems[:NBUF]
    wsems = sems[NBUF:]
    wid = lax.axis_index("s") * NUM_CORES + lax.axis_index("c")
    col0 = wid * COLS_PER_W
    pltpu.sync_copy(idx_hbm.at[:, pl.ds(col0, COLS_PER_W)], idx_v)

    def gather_start(s, b):
        pltpu.async_copy(table_hbm.at[idx_v.at[s]], row_v.at[b], gsems[b])

    def gather_wait(b):
        pltpu.make_async_copy(
            table_hbm.at[idx_v.at[0]], row_v.at[b], gsems[b]
        ).wait()

    def write_start(s, ob):
        for fh in range(8):
            pltpu.async_copy(
                out_v.at[ob, pl.ds(8 * fh, 8)], out_hbm.at[s, fh, wid], wsems[ob]
            )

    def write_wait(ob):
        for fh in range(8):
            pltpu.make_async_copy(
                out_v.at[ob, pl.ds(8 * fh, 8)], out_hbm.at[0, fh, wid], wsems[ob]
            ).wait()

    iota = lax.iota(jnp.int32, 16)
    perms = [(iota + d) & 15 for d in range(16)]

    def transpose(b, ob):
        src = row_v.at[b]
        dst = out_v.at[ob]

        @plsc.parallel_loop(0, 32, 1, unroll=2)
        def _(blk):
            t0 = (blk >> 2) * 16
            f0 = (blk & 3) * 16
            rowr = iota + t0
            for d in range(16):
                colr = perms[d] + f0
                x = plsc.load_gather(src, [rowr, colr])
                plsc.store_scatter(dst, [colr, rowr], x)

    for b in range(NBUF):
        gather_start(b, b)

    def group(g, carry):
        for k in range(INNER):
            s = g * INNER + k
            b = k % NBUF
            ob = k % NOUT
            gather_wait(b)

            @pl.when(s >= NOUT)
            def _():
                write_wait(ob)

            transpose(b, ob)

            @pl.when(s < NSEQ - NBUF)
            def _():
                gather_start(s + NBUF, b)

            write_start(s, ob)
        return carry

    lax.fori_loop(0, NGROUPS, group, 0)
    for ob in range(NOUT):
        write_wait(ob)


def kernel(input, weight):
    mesh = plsc.VectorSubcoreMesh(core_axis_name="c", subcore_axis_name="s")

    wt = weight.T
    k1 = functools.partial(
        pl.kernel,
        mesh=mesh,
        out_type=jax.ShapeDtypeStruct((NUM_EMBEDDINGS // 2, 128), jnp.float32),
        scratch_types=[
            pltpu.VMEM((RING, EMBEDDING_DIM, SLAB_V), jnp.float32),
            pltpu.VMEM((RING, SLAB_V // 2, 128), jnp.float32),
        ]
        + [pltpu.SemaphoreType.DMA] * (2 * RING),
        compiler_params=pltpu.CompilerParams(needs_layout_passes=False),
    )(_transpose_body)
    tail128 = jnp.concatenate(
        [wt[:, NSLAB * SLAB_V:], jnp.zeros((EMBEDDING_DIM, 128 - TAIL_V), jnp.float32)],
        axis=1,
    )
    pair_table = k1(wt, tail128)
    table = pair_table.reshape(NUM_EMBEDDINGS, EMBEDDING_DIM)

    idx_t = input.T
    k2 = functools.partial(
        pl.kernel,
        mesh=mesh,
        out_type=jax.ShapeDtypeStruct(
            (NSEQ, EMBEDDING_DIM // 8, NW, 8, COLS_PER_W), jnp.float32
        ),
        scratch_types=[
            pltpu.VMEM((NSEQ, COLS_PER_W), jnp.int32),
            pltpu.VMEM((NBUF, COLS_PER_W, EMBEDDING_DIM), jnp.float32),
            pltpu.VMEM((NOUT, EMBEDDING_DIM, COLS_PER_W), jnp.float32),
        ]
        + [pltpu.SemaphoreType.DMA] * (NBUF + NOUT),
        compiler_params=pltpu.CompilerParams(
            use_tc_tiling_on_sc=False, needs_layout_passes=False
        ),
    )(_gather_body)
    out5 = k2(idx_t, table)
    return out5.transpose(2, 4, 0, 1, 3).reshape(NBATCH, NSEQ, EMBEDDING_DIM)


# speedup vs baseline: 1.0089x; 1.0089x over previous
import functools

import jax
import jax.numpy as jnp
from jax import lax
from jax.experimental import pallas as pl
from jax.experimental.pallas import tpu as pltpu
from jax.experimental.pallas import tpu_sc as plsc

NUM_EMBEDDINGS = 1000000
EMBEDDING_DIM = 64

NBATCH = 4096
NSEQ = 50
NUM_CORES = 2
NUM_SUBCORES = 16
NW = NUM_CORES * NUM_SUBCORES

SLAB_V = 256
NSLAB = NUM_EMBEDDINGS // SLAB_V
TAIL_V = NUM_EMBEDDINGS - NSLAB * SLAB_V
RUNROLL = 2


def _shuffle(src, out_v, ok, nrows):
    iota = lax.iota(jnp.int32, 16)
    perms = [(iota + d) & 15 for d in range(16)]
    out2d = out_v.at[ok]

    @plsc.parallel_loop(0, (nrows // 16) * 8, 1, unroll=RUNROLL)
    def _(blk):
        rg = blk >> 3
        fg = (blk >> 1) & 3
        p = blk & 1
        r0 = rg * 16
        f0 = fg * 16
        rowr = iota + f0
        colw = iota + (p * 64 + f0)
        base = 2 * r0 + p
        for d in range(16):
            colr = 2 * perms[d] + base
            x = plsc.load_gather(src, [rowr, colr])
            plsc.store_scatter(out2d, [perms[d] + r0, colw], x)


RING = 3


def _transpose_body(wt_hbm, tail_hbm, wp_hbm, in_v, out_v, *sems):
    isems = sems[:RING]
    osems = sems[RING:]
    wid = lax.axis_index("s") * NUM_CORES + lax.axis_index("c")

    def in_start(j, k):
        pltpu.async_copy(
            wt_hbm.at[:, pl.ds(j * SLAB_V, SLAB_V)], in_v.at[k], isems[k]
        )

    def in_wait(k):
        pltpu.make_async_copy(
            wt_hbm.at[:, pl.ds(0, SLAB_V)], in_v.at[k], isems[k]
        ).wait()

    def out_start(j, k):
        pltpu.async_copy(
            out_v.at[k], wp_hbm.at[pl.ds(j * (SLAB_V // 2), SLAB_V // 2)], osems[k]
        )

    def out_wait(k):
        pltpu.make_async_copy(
            out_v.at[k], wp_hbm.at[pl.ds(0, SLAB_V // 2)], osems[k]
        ).wait()

    def do_slab(j, k, first, last):
        in_wait(k)

        @pl.when(jnp.logical_not(first))
        def _():
            out_wait(k)

        _shuffle(in_v.at[k], out_v, k, SLAB_V // 2)

        @pl.when(jnp.logical_not(last))
        def _():
            in_start(j + RING * NW, k)

        out_start(j, k)

    nmine = (NSLAB - 1 - wid) // NW + 1

    for k in range(RING):
        in_start(wid + k * NW, k)

    def group(i, carry):
        for k in range(RING):
            idx = RING * i + k

            @pl.when(idx < nmine)
            def _():
                do_slab(wid + idx * NW, k, idx < RING, idx + RING >= nmine)

        return carry

    lax.fori_loop(0, (nmine + RING - 1) // RING, group, 0)
    for k in range(RING):
        out_wait(k)

    @pl.when(wid == 0)
    def _():
        pltpu.sync_copy(tail_hbm, in_v.at[0, :, pl.ds(0, 128)])
        _shuffle(in_v.at[0], out_v, 0, TAIL_V // 2)
        pltpu.sync_copy(
            out_v.at[0, pl.ds(0, TAIL_V // 2)],
            wp_hbm.at[pl.ds(NSLAB * SLAB_V // 2, TAIL_V // 2)],
        )


COLS_PER_W = NBATCH // NW
NBUF = 5
NOUT = 5
INNER = 5
NGROUPS = NSEQ // INNER


def _gather_body(idx_hbm, table_hbm, out_hbm, idx_v, row_v, out_v, *sems):
    gsems = sems[:NBUF]
    wsems = sems[NBUF:]
    wid = lax.axis_index("s") * NUM_CORES + lax.axis_index("c")
    col0 = wid * COLS_PER_W
    pltpu.sync_copy(idx_hbm.at[:, pl.ds(col0, COLS_PER_W)], idx_v)

    def gather_start(s, b):
        pltpu.async_copy(table_hbm.at[idx_v.at[s]], row_v.at[b], gsems[b])

    def gather_wait(b):
        pltpu.make_async_copy(
            table_hbm.at[idx_v.at[0]], row_v.at[b], gsems[b]
        ).wait()

    def write_start(s, ob):
        for fh in range(8):
            pltpu.async_copy(
                out_v.at[ob, pl.ds(8 * fh, 8)], out_hbm.at[s, fh, wid], wsems[ob]
            )

    def write_wait(ob):
        for fh in range(8):
            pltpu.make_async_copy(
                out_v.at[ob, pl.ds(8 * fh, 8)], out_hbm.at[0, fh, wid], wsems[ob]
            ).wait()

    iota = lax.iota(jnp.int32, 16)
    perms = [(iota + d) & 15 for d in range(16)]

    def transpose(b, ob):
        src = row_v.at[b]
        dst = out_v.at[ob]

        @plsc.parallel_loop(0, 32, 1, unroll=2)
        def _(blk):
            t0 = (blk >> 2) * 16
            f0 = (blk & 3) * 16
            rowr = iota + t0
            for d in range(16):
                colr = perms[d] + f0
                x = plsc.load_gather(src, [rowr, colr])
                plsc.store_scatter(dst, [colr, rowr], x)

    for b in range(NBUF):
        gather_start(b, b)

    def group(g, carry):
        for k in range(INNER):
            s = g * INNER + k
            b = k % NBUF
            ob = k % NOUT
            gather_wait(b)

            @pl.when(s >= NOUT)
            def _():
                write_wait(ob)

            transpose(b, ob)

            @pl.when(s < NSEQ - NBUF)
            def _():
                gather_start(s + NBUF, b)

            write_start(s, ob)
        return carry

    lax.fori_loop(0, NGROUPS, group, 0)
    for ob in range(NOUT):
        write_wait(ob)


def kernel(input, weight):
    mesh = plsc.VectorSubcoreMesh(core_axis_name="c", subcore_axis_name="s")

    wt = weight.T
    k1 = functools.partial(
        pl.kernel,
        mesh=mesh,
        out_type=jax.ShapeDtypeStruct((NUM_EMBEDDINGS // 2, 128), jnp.float32),
        scratch_types=[
            pltpu.VMEM((RING, EMBEDDING_DIM, SLAB_V), jnp.float32),
            pltpu.VMEM((RING, SLAB_V // 2, 128), jnp.float32),
        ]
        + [pltpu.SemaphoreType.DMA] * (2 * RING),
        compiler_params=pltpu.CompilerParams(needs_layout_passes=False),
    )(_transpose_body)
    tail128 = jnp.concatenate(
        [wt[:, NSLAB * SLAB_V:], jnp.zeros((EMBEDDING_DIM, 128 - TAIL_V), jnp.float32)],
        axis=1,
    )
    pair_table = k1(wt, tail128)
    table = pair_table.reshape(NUM_EMBEDDINGS, EMBEDDING_DIM)

    idx_t = input.T
    k2 = functools.partial(
        pl.kernel,
        mesh=mesh,
        out_type=jax.ShapeDtypeStruct(
            (NSEQ, EMBEDDING_DIM // 8, NW, 8, COLS_PER_W), jnp.float32
        ),
        scratch_types=[
            pltpu.VMEM((NSEQ, COLS_PER_W), jnp.int32),
            pltpu.VMEM((NBUF, COLS_PER_W, EMBEDDING_DIM), jnp.float32),
            pltpu.VMEM((NOUT, EMBEDDING_DIM, COLS_PER_W), jnp.float32),
        ]
        + [pltpu.SemaphoreType.DMA] * (NBUF + NOUT),
        compiler_params=pltpu.CompilerParams(
            use_tc_tiling_on_sc=False, needs_layout_passes=False
        ),
    )(_gather_body)
    out5 = k2(idx_t, table)
    return out5.transpose(2, 4, 0, 1, 3).reshape(NBATCH, NSEQ, EMBEDDING_DIM)
